# baseline (speedup 1.0000x reference)
import jax
import jax.numpy as jnp
from jax import lax
from jax.experimental import pallas as pl
from jax.experimental.pallas import tpu as pltpu

_DIM = 512
_NUM_ITEM = 4096
_K = 16
_TR = 512


def _tile_body(x_ref, mp_ref, mpb_ref, out1_ref, out2_ref):
    qc = x_ref[0]
    att = lax.dot_general(
        qc,
        mp_ref[...],
        (((0,), (1,)), ((), ())),
        preferred_element_type=jnp.float32,
    )
    nsl = _NUM_ITEM // 128
    sl = [att[:, 128 * g : 128 * (g + 1)] for g in range(nsl)]
    def _cs(x, y):
        return jnp.maximum(x, y), jnp.minimum(x, y)

    def _sort4(w, x, y, z):
        w, x = _cs(w, x)
        y, z = _cs(y, z)
        w, y = _cs(w, y)
        x, z = _cs(x, z)
        x, y = _cs(x, y)
        return w, x, y, z

    def _merge_top4(a, b, cleanup=True):
        c1 = jnp.maximum(a[0], b[3])
        c2 = jnp.maximum(a[1], b[2])
        c3 = jnp.maximum(a[2], b[1])
        c4 = jnp.maximum(a[3], b[0])
        if cleanup:
            c1, c3 = _cs(c1, c3)
            c2, c4 = _cs(c2, c4)
            c1, c2 = _cs(c1, c2)
            c3, c4 = _cs(c3, c4)
        return c1, c2, c3, c4

    groups = [
        _sort4(sl[i], sl[i + 1], sl[i + 2], sl[i + 3])
        for i in range(0, nsl, 4)
    ]
    while len(groups) > 2:
        groups = [
            _merge_top4(groups[i], groups[i + 1])
            for i in range(0, len(groups), 2)
        ]
    top4 = _merge_top4(groups[0], groups[1], cleanup=False)
    gcat = jnp.concatenate(top4, axis=1)
    m0 = jnp.max(gcat, axis=1, keepdims=True)
    m = m0
    denom = jnp.exp(m0)
    for _ in range(_K - 1):
        m = jnp.max(
            jnp.where(gcat < m, gcat, -jnp.inf), axis=1, keepdims=True
        )
        denom = denom + jnp.exp(m)
    t = m
    recip = 1.0 / denom
    pn = jnp.where(att >= t, jnp.exp(att) * recip, 0.0)
    out2_ref[...] = pn * att
    out1t = lax.dot_general(
        mpb_ref[...],
        pn.astype(jnp.bfloat16),
        (((0,), (1,)), ((), ())),
        preferred_element_type=jnp.float32,
    )
    out1_ref[0] = out1t


def kernel(input, mempool):
    B, C, H, W = input.shape
    x3 = input.reshape(B, C, H * W)
    rows = B * H * W
    ntiles = rows // _TR
    tpb = (H * W) // _TR
    out1, out2 = pl.pallas_call(
        _tile_body,
        grid=(ntiles,),
        in_specs=[
            pl.BlockSpec((1, C, _TR), lambda i: (i // tpb, 0, i % tpb)),
            pl.BlockSpec((_NUM_ITEM, C), lambda i: (0, 0)),
            pl.BlockSpec((_NUM_ITEM, C), lambda i: (0, 0)),
        ],
        out_specs=[
            pl.BlockSpec((1, C, _TR), lambda i: (i // tpb, 0, i % tpb)),
            pl.BlockSpec((_TR, _NUM_ITEM), lambda i: (i, 0)),
        ],
        out_shape=[
            jax.ShapeDtypeStruct((B, C, H * W), jnp.float32),
            jax.ShapeDtypeStruct((rows, _NUM_ITEM), jnp.float32),
        ],
        compiler_params=pltpu.CompilerParams(
            dimension_semantics=("arbitrary",),
        ),
    )(x3, mempool, mempool.astype(jnp.bfloat16))
    return out1.reshape(B, C, H, W), out2


# speedup vs baseline: 1.0161x; 1.0161x over previous
import jax
import jax.numpy as jnp
from jax import lax
from jax.experimental import pallas as pl
from jax.experimental.pallas import tpu as pltpu

_DIM = 512
_NUM_ITEM = 4096
_K = 16
_TR = 512


def _tile_body(x_ref, mp_ref, mpb_ref, out1_ref, out2_ref):
    qc = x_ref[0]
    att = lax.dot_general(
        qc,
        mp_ref[...],
        (((0,), (1,)), ((), ())),
        preferred_element_type=jnp.float32,
    )
    nsl = _NUM_ITEM // 128
    sl = [att[:, 128 * g : 128 * (g + 1)] for g in range(nsl)]
    def _cs(x, y):
        return jnp.maximum(x, y), jnp.minimum(x, y)

    def _sort4(w, x, y, z):
        w, x = _cs(w, x)
        y, z = _cs(y, z)
        w, y = _cs(w, y)
        x, z = _cs(x, z)
        x, y = _cs(x, y)
        return w, x, y, z

    def _merge_top4(a, b, cleanup=True):
        c1 = jnp.maximum(a[0], b[3])
        c2 = jnp.maximum(a[1], b[2])
        c3 = jnp.maximum(a[2], b[1])
        c4 = jnp.maximum(a[3], b[0])
        if cleanup:
            c1, c3 = _cs(c1, c3)
            c2, c4 = _cs(c2, c4)
            c1, c2 = _cs(c1, c2)
            c3, c4 = _cs(c3, c4)
        return c1, c2, c3, c4

    groups = [
        _sort4(sl[i], sl[i + 1], sl[i + 2], sl[i + 3])
        for i in range(0, nsl, 4)
    ]
    while len(groups) > 2:
        groups = [
            _merge_top4(groups[i], groups[i + 1])
            for i in range(0, len(groups), 2)
        ]
    top4 = _merge_top4(groups[0], groups[1], cleanup=False)
    gcat = jnp.concatenate(top4, axis=1)
    m0 = jnp.max(gcat, axis=1, keepdims=True)
    m = m0
    denom = jnp.ones((_TR, 1), jnp.float32)
    for _ in range(_K - 1):
        m = jnp.max(
            jnp.where(gcat < m, gcat, -jnp.inf), axis=1, keepdims=True
        )
        denom = denom + jnp.exp(m - m0)
    t = m
    p = jnp.where(att >= t, jnp.exp(att - m0), 0.0)
    recip = 1.0 / denom
    out2_ref[...] = p * att * recip
    out1t = lax.dot_general(
        mpb_ref[...],
        p.astype(jnp.bfloat16),
        (((0,), (1,)), ((), ())),
        preferred_element_type=jnp.float32,
    )
    out1_ref[0] = out1t * jnp.reshape(recip, (1, _TR))


def kernel(input, mempool):
    B, C, H, W = input.shape
    x3 = input.reshape(B, C, H * W)
    rows = B * H * W
    ntiles = rows // _TR
    tpb = (H * W) // _TR
    out1, out2 = pl.pallas_call(
        _tile_body,
        grid=(ntiles,),
        in_specs=[
            pl.BlockSpec((1, C, _TR), lambda i: (i // tpb, 0, i % tpb)),
            pl.BlockSpec((_NUM_ITEM, C), lambda i: (0, 0)),
            pl.BlockSpec((_NUM_ITEM, C), lambda i: (0, 0)),
        ],
        out_specs=[
            pl.BlockSpec((1, C, _TR), lambda i: (i // tpb, 0, i % tpb)),
            pl.BlockSpec((_TR, _NUM_ITEM), lambda i: (i, 0)),
        ],
        out_shape=[
            jax.ShapeDtypeStruct((B, C, H * W), jnp.float32),
            jax.ShapeDtypeStruct((rows, _NUM_ITEM), jnp.float32),
        ],
        compiler_params=pltpu.CompilerParams(
            dimension_semantics=("arbitrary",),
        ),
    )(x3, mempool, mempool.astype(jnp.bfloat16))
    return out1.reshape(B, C, H, W), out2


# speedup vs baseline: 1.0218x; 1.0056x over previous
import jax
import jax.numpy as jnp
from jax import lax
from jax.experimental import pallas as pl
from jax.experimental.pallas import tpu as pltpu

_DIM = 512
_NUM_ITEM = 4096
_K = 16
_TR = 512


def _tile_body(x_ref, mp_ref, mpb_ref, out1_ref, out2_ref):
    qc = x_ref[0]
    att = lax.dot_general(
        qc,
        mp_ref[...],
        (((0,), (1,)), ((), ())),
        preferred_element_type=jnp.float32,
    )
    nsl = _NUM_ITEM // 128
    sl = [att[:, 128 * g : 128 * (g + 1)] for g in range(nsl)]
    def _cs(x, y):
        return jnp.maximum(x, y), jnp.minimum(x, y)

    def _sort4(w, x, y, z):
        w, x = _cs(w, x)
        y, z = _cs(y, z)
        w, y = _cs(w, y)
        x, z = _cs(x, z)
        x, y = _cs(x, y)
        return w, x, y, z

    def _merge_top4(a, b, cleanup=True):
        c1 = jnp.maximum(a[0], b[3])
        c2 = jnp.maximum(a[1], b[2])
        c3 = jnp.maximum(a[2], b[1])
        c4 = jnp.maximum(a[3], b[0])
        if cleanup:
            c1, c3 = _cs(c1, c3)
            c2, c4 = _cs(c2, c4)
            c1, c2 = _cs(c1, c2)
            c3, c4 = _cs(c3, c4)
        return c1, c2, c3, c4

    groups = [
        _sort4(sl[i], sl[i + 1], sl[i + 2], sl[i + 3])
        for i in range(0, nsl, 4)
    ]
    while len(groups) > 2:
        groups = [
            _merge_top4(groups[i], groups[i + 1])
            for i in range(0, len(groups), 2)
        ]
    top4 = _merge_top4(groups[0], groups[1], cleanup=False)
    gcat = jnp.concatenate(top4, axis=1)
    m0 = jnp.max(gcat, axis=1, keepdims=True)
    m = m0
    denom = jnp.ones((_TR, 1), jnp.float32)
    for _ in range((_K - 1) // 3):
        g1 = jnp.where(gcat < m, gcat, -jnp.inf)
        ma = jnp.max(g1, axis=1, keepdims=True)
        g2 = jnp.where(gcat < ma, g1, -jnp.inf)
        mb = jnp.max(g2, axis=1, keepdims=True)
        mc = jnp.max(
            jnp.where(gcat < mb, g2, -jnp.inf), axis=1, keepdims=True
        )
        denom = denom + jnp.exp(ma - m0) + jnp.exp(mb - m0) + jnp.exp(mc - m0)
        m = mc
    t = m
    p = jnp.where(att >= t, jnp.exp(att - m0), 0.0)
    recip = 1.0 / denom
    out2_ref[...] = p * att * recip
    out1t = lax.dot_general(
        mpb_ref[...],
        p.astype(jnp.bfloat16),
        (((0,), (1,)), ((), ())),
        preferred_element_type=jnp.float32,
    )
    out1_ref[0] = out1t * jnp.reshape(recip, (1, _TR))


def kernel(input, mempool):
    B, C, H, W = input.shape
    x3 = input.reshape(B, C, H * W)
    rows = B * H * W
    ntiles = rows // _TR
    tpb = (H * W) // _TR
    out1, out2 = pl.pallas_call(
        _tile_body,
        grid=(ntiles,),
        in_specs=[
            pl.BlockSpec((1, C, _TR), lambda i: (i // tpb, 0, i % tpb)),
            pl.BlockSpec((_NUM_ITEM, C), lambda i: (0, 0)),
            pl.BlockSpec((_NUM_ITEM, C), lambda i: (0, 0)),
        ],
        out_specs=[
            pl.BlockSpec((1, C, _TR), lambda i: (i // tpb, 0, i % tpb)),
            pl.BlockSpec((_TR, _NUM_ITEM), lambda i: (i, 0)),
        ],
        out_shape=[
            jax.ShapeDtypeStruct((B, C, H * W), jnp.float32),
            jax.ShapeDtypeStruct((rows, _NUM_ITEM), jnp.float32),
        ],
        compiler_params=pltpu.CompilerParams(
            dimension_semantics=("arbitrary",),
        ),
    )(x3, mempool, mempool.astype(jnp.bfloat16))
    return out1.reshape(B, C, H, W), out2
